# Initial kernel scaffold; baseline (speedup 1.0000x reference)
import jax
import jax.numpy as jnp
from jax.experimental import pallas as pl


def kernel(x, h0_a1, h0_a2, h0_b2, h1_a1, h1_a2, h1_b2, Wm, bm):
    raise NotImplementedError("write your pallas kernel here")


# speedup vs baseline: 71.3149x; 71.3149x over previous
import jax
import jax.numpy as jnp
from jax.experimental import pallas as pl

_B = 128
_N = 8192
_C = 25
_K = 5
_BT = 8


def _main_kernel(x_ref, a1_ref, a2_ref, b2_ref, v_ref):
    for b in range(_BT):
        cols = []
        for br in range(2):
            a1 = a1_ref[br]
            a2 = a2_ref[br]
            b2 = b2_ref[br]
            x0 = x_ref[br, b, 0:1, :]
            x1 = x_ref[br, b, 1:2, :]
            h = a1[:, 0:1] * x0 + a1[:, 1:2] * x1
            m = jnp.max(h, axis=1, keepdims=True)
            res = (jnp.dot(a2, h, preferred_element_type=jnp.float32)
                   + jnp.dot(b2, m, preferred_element_type=jnp.float32))
            iota = jax.lax.broadcasted_iota(jnp.int32, res.shape, 1)
            r = res
            for _ in range(_K):
                mx = jnp.max(r, axis=1, keepdims=True)
                cols.append(mx)
                first = jnp.min(jnp.where(r == mx, iota, _N), axis=1,
                                keepdims=True)
                r = jnp.where(iota == first, -jnp.inf, r)
        v_ref[b] = jnp.concatenate(cols, axis=1)


def _head_kernel(v_ref, w_ref, b_ref, o_ref):
    o_ref[...] = (jnp.dot(v_ref[...], w_ref[...],
                          preferred_element_type=jnp.float32)
                  + b_ref[...])


def kernel(x, h0_a1, h0_a2, h0_b2, h1_a1, h1_a2, h1_b2, Wm, bm):
    A1 = jnp.stack((h0_a1, h1_a1))
    A2 = jnp.stack((h0_a2, h1_a2))
    B2 = jnp.stack((h0_b2, h1_b2))

    v = pl.pallas_call(
        _main_kernel,
        grid=(_B // _BT,),
        in_specs=[
            pl.BlockSpec((2, _BT, 2, _N), lambda i: (0, i, 0, 0)),
            pl.BlockSpec((2, _C, 2), lambda i: (0, 0, 0)),
            pl.BlockSpec((2, _C, _C), lambda i: (0, 0, 0)),
            pl.BlockSpec((2, _C, _C), lambda i: (0, 0, 0)),
        ],
        out_specs=pl.BlockSpec((_BT, _C, 2 * _K), lambda i: (i, 0, 0)),
        out_shape=jax.ShapeDtypeStruct((_B, _C, 2 * _K), jnp.float32),
    )(x, A1, A2, B2)

    Wx = (Wm.reshape(5, 2, _C, _K)
            .transpose(0, 2, 1, 3)
            .reshape(5, _C * 2 * _K))
    v2 = v.reshape(_B, _C * 2 * _K)

    logits = pl.pallas_call(
        _head_kernel,
        in_specs=[
            pl.BlockSpec((_B, _C * 2 * _K), lambda: (0, 0)),
            pl.BlockSpec((_C * 2 * _K, 5), lambda: (0, 0)),
            pl.BlockSpec((1, 5), lambda: (0, 0)),
        ],
        out_specs=pl.BlockSpec((_B, 5), lambda: (0, 0)),
        out_shape=jax.ShapeDtypeStruct((_B, 5), jnp.float32),
    )(v2, Wx.T, bm.reshape(1, 5))
    return logits


# speedup vs baseline: 76.6434x; 1.0747x over previous
import jax
import jax.numpy as jnp
from jax.experimental import pallas as pl

_B = 128
_N = 8192
_C = 25
_K = 5
_BT = 8


def _main_kernel(x_ref, a1_ref, a2_ref, b2_ref, v_ref):
    iota = jax.lax.broadcasted_iota(jnp.int32, (_C, _N), 1)
    oh = [(jax.lax.broadcasted_iota(jnp.int32, (1, 2 * _K), 1) == c
           ).astype(jnp.float32) for c in range(2 * _K)]
    a1s = [a1_ref[br] for br in range(2)]
    a2s = [a2_ref[br] for br in range(2)]
    b2s = [b2_ref[br] for br in range(2)]
    for b in range(_BT):
        v = jnp.zeros((_C, 2 * _K), jnp.float32)
        for br in range(2):
            a1, a2, b2 = a1s[br], a2s[br], b2s[br]
            x0 = x_ref[br, b, 0:1, :]
            x1 = x_ref[br, b, 1:2, :]
            h = a1[:, 0:1] * x0 + a1[:, 1:2] * x1
            m = jnp.max(h, axis=1, keepdims=True)
            res = (jnp.dot(a2, h, preferred_element_type=jnp.float32)
                   + jnp.dot(b2, m, preferred_element_type=jnp.float32))
            r = res
            for t in range(_K):
                mx = jnp.max(r, axis=1, keepdims=True)
                v = v + mx * oh[br * _K + t]
                first = jnp.min(jnp.where(r == mx, iota, _N), axis=1,
                                keepdims=True)
                r = jnp.where(iota == first, -jnp.inf, r)
        v_ref[b] = v


def _head_kernel(v_ref, w_ref, b_ref, o_ref):
    o_ref[...] = (jnp.dot(v_ref[...], w_ref[...],
                          preferred_element_type=jnp.float32)
                  + b_ref[...])


def kernel(x, h0_a1, h0_a2, h0_b2, h1_a1, h1_a2, h1_b2, Wm, bm):
    A1 = jnp.stack((h0_a1, h1_a1))
    A2 = jnp.stack((h0_a2, h1_a2))
    B2 = jnp.stack((h0_b2, h1_b2))

    v = pl.pallas_call(
        _main_kernel,
        grid=(_B // _BT,),
        in_specs=[
            pl.BlockSpec((2, _BT, 2, _N), lambda i: (0, i, 0, 0)),
            pl.BlockSpec((2, _C, 2), lambda i: (0, 0, 0)),
            pl.BlockSpec((2, _C, _C), lambda i: (0, 0, 0)),
            pl.BlockSpec((2, _C, _C), lambda i: (0, 0, 0)),
        ],
        out_specs=pl.BlockSpec((_BT, _C, 2 * _K), lambda i: (i, 0, 0)),
        out_shape=jax.ShapeDtypeStruct((_B, _C, 2 * _K), jnp.float32),
    )(x, A1, A2, B2)

    Wx = (Wm.reshape(5, 2, _C, _K)
            .transpose(0, 2, 1, 3)
            .reshape(5, _C * 2 * _K))
    v2 = v.reshape(_B, _C * 2 * _K)

    logits = pl.pallas_call(
        _head_kernel,
        in_specs=[
            pl.BlockSpec((_B, _C * 2 * _K), lambda: (0, 0)),
            pl.BlockSpec((_C * 2 * _K, 5), lambda: (0, 0)),
            pl.BlockSpec((1, 5), lambda: (0, 0)),
        ],
        out_specs=pl.BlockSpec((_B, 5), lambda: (0, 0)),
        out_shape=jax.ShapeDtypeStruct((_B, 5), jnp.float32),
    )(v2, Wx.T, bm.reshape(1, 5))
    return logits


# speedup vs baseline: 134.1659x; 1.7505x over previous
import jax
import jax.numpy as jnp
from jax.experimental import pallas as pl

_B = 128
_N = 8192
_C = 25
_K = 5
_BT = 8
_LANES = 128


def _merge(A, B, cap=_K):
    la, lb = len(A), len(B)
    out = []
    for j in range(min(la + lb, cap)):
        terms = []
        if j < lb:
            terms.append(B[j])
        if j < la:
            terms.append(A[j])
        for i in range(1, j + 1):
            if i - 1 < la and j - i < lb:
                terms.append(jnp.minimum(A[i - 1], B[j - i]))
        acc = terms[0]
        for t in terms[1:]:
            acc = jnp.maximum(acc, t)
        out.append(acc)
    return out


def _top5_rows(q):
    level = [[q[:, c * _LANES:(c + 1) * _LANES]] for c in range(_N // _LANES)]
    while len(level) > 1:
        level = [_merge(level[2 * i], level[2 * i + 1])
                 for i in range(len(level) // 2)]
    T = level[0]
    s = _LANES // 2
    while s >= 1:
        T = _merge(T, [jnp.roll(t, s, axis=1) for t in T])
        s //= 2
    return [t[:, 0:1] for t in T]


def _main_kernel(x_ref, wu_ref, b2p_ref, v_ref):
    oh = [(jax.lax.broadcasted_iota(jnp.int32, (1, 2 * _K), 1) == c
           ).astype(jnp.float32) for c in range(2 * _K)]
    brmask = [sum(oh[br * _K:(br + 1) * _K][1:], oh[br * _K]) for br in (0, 1)]
    wus = [wu_ref[br] for br in range(2)]
    b2ps = [b2p_ref[br] for br in range(2)]
    for b in range(_BT):
        v = jnp.zeros((32, 2 * _K), jnp.float32)
        for br in range(2):
            x2 = x_ref[br, b]
            g = jnp.dot(wus[br], x2,
                        preferred_element_type=jnp.float32)
            h = g[0:32, :]
            q = g[32:64, :]
            m = jnp.max(h, axis=1, keepdims=True)
            bias = jnp.dot(b2ps[br], m,
                           preferred_element_type=jnp.float32)
            tops = _top5_rows(q)
            for t in range(_K):
                v = v + tops[t] * oh[br * _K + t]
            v = v + bias * brmask[br]
        v_ref[b] = v


def _head_kernel(v_ref, w_ref, b_ref, o_ref):
    o_ref[...] = (jnp.dot(v_ref[...], w_ref[...],
                          preferred_element_type=jnp.float32)
                  + b_ref[...])


def kernel(x, h0_a1, h0_a2, h0_b2, h1_a1, h1_a2, h1_b2, Wm, bm):
    z7 = jnp.zeros((7, 2), jnp.float32)
    wu0 = jnp.concatenate((h0_a1, z7, h0_a2 @ h0_a1, z7), axis=0)
    wu1 = jnp.concatenate((h1_a1, z7, h1_a2 @ h1_a1, z7), axis=0)
    WU = jnp.stack((wu0, wu1))
    B2P = jnp.zeros((2, 32, 32), jnp.float32)
    B2P = B2P.at[0, :_C, :_C].set(h0_b2).at[1, :_C, :_C].set(h1_b2)

    v = pl.pallas_call(
        _main_kernel,
        grid=(_B // _BT,),
        in_specs=[
            pl.BlockSpec((2, _BT, 2, _N), lambda i: (0, i, 0, 0)),
            pl.BlockSpec((2, 64, 2), lambda i: (0, 0, 0)),
            pl.BlockSpec((2, 32, 32), lambda i: (0, 0, 0)),
        ],
        out_specs=pl.BlockSpec((_BT, 32, 2 * _K), lambda i: (i, 0, 0)),
        out_shape=jax.ShapeDtypeStruct((_B, 32, 2 * _K), jnp.float32),
    )(x, WU, B2P)

    Wx = (Wm.reshape(5, 2, _C, _K)
            .transpose(0, 2, 1, 3)
            .reshape(5, _C * 2 * _K))
    v2 = v[:, :_C, :].reshape(_B, _C * 2 * _K)

    logits = pl.pallas_call(
        _head_kernel,
        in_specs=[
            pl.BlockSpec((_B, _C * 2 * _K), lambda: (0, 0)),
            pl.BlockSpec((_C * 2 * _K, 5), lambda: (0, 0)),
            pl.BlockSpec((1, 5), lambda: (0, 0)),
        ],
        out_specs=pl.BlockSpec((_B, 5), lambda: (0, 0)),
        out_shape=jax.ShapeDtypeStruct((_B, 5), jnp.float32),
    )(v2, Wx.T, bm.reshape(1, 5))
    return logits


# speedup vs baseline: 146.3891x; 1.0911x over previous
import jax
import jax.numpy as jnp
from jax.experimental import pallas as pl

_B = 128
_N = 8192
_C = 25
_K = 5
_BT = 8
_R = _BT * _C
_LANES = 128


def _merge(A, B, cap=_K):
    la, lb = len(A), len(B)
    out = []
    for j in range(min(la + lb, cap)):
        terms = []
        if j < lb:
            terms.append(B[j])
        if j < la:
            terms.append(A[j])
        for i in range(1, j + 1):
            if i - 1 < la and j - i < lb:
                terms.append(jnp.minimum(A[i - 1], B[j - i]))
        acc = terms[0]
        for t in terms[1:]:
            acc = jnp.maximum(acc, t)
        out.append(acc)
    return out


def _pairtree(xs, op):
    while len(xs) > 1:
        nxt = [op(xs[2 * i], xs[2 * i + 1]) for i in range(len(xs) // 2)]
        if len(xs) % 2:
            nxt.append(xs[-1])
        xs = nxt
    return xs[0]


def _top5_rows(q):
    level = [[q[:, c * _LANES:(c + 1) * _LANES]] for c in range(_N // _LANES)]
    while len(level) > 1:
        level = [_merge(level[2 * i], level[2 * i + 1])
                 for i in range(len(level) // 2)]
    T = level[0]
    s = _LANES // 2
    while s >= 1:
        T = _merge(T, [jnp.roll(t, s, axis=1) for t in T])
        s //= 2
    return [t[:, 0:1] for t in T]


def _main_kernel(x_ref, w_ref, b2k_ref, v_ref):
    oh = [(jax.lax.broadcasted_iota(jnp.int32, (1, _K), 1) == c
           ).astype(jnp.float32) for c in range(_K)]
    for br in range(2):
        x16 = x_ref[br]
        g = jnp.dot(w_ref[br], x16,
                    preferred_element_type=jnp.float32)
        h = g[0:_R, :]
        q = g[_R:2 * _R, :]
        hm = _pairtree([h[:, c * _LANES:(c + 1) * _LANES]
                        for c in range(_N // _LANES)], jnp.maximum)
        m = jnp.max(hm, axis=1, keepdims=True)
        bias = jnp.dot(b2k_ref[br], m,
                       preferred_element_type=jnp.float32)
        tops = _top5_rows(q)
        v = bias + tops[0] * oh[0]
        for t in range(1, _K):
            v = v + tops[t] * oh[t]
        v_ref[br, 0] = v


def _head_kernel(v_ref, w_ref, b_ref, o_ref):
    o_ref[...] = (jnp.dot(v_ref[...], w_ref[...],
                          preferred_element_type=jnp.float32)
                  + b_ref[...])


def kernel(x, h0_a1, h0_a2, h0_b2, h1_a1, h1_a2, h1_b2, Wm, bm):
    W = jnp.zeros((2, 2 * _R, 2 * _BT), jnp.float32)
    B2K = jnp.zeros((2, _R, _R), jnp.float32)
    for br, (a1, a2, b2) in enumerate(((h0_a1, h0_a2, h0_b2),
                                       (h1_a1, h1_a2, h1_b2))):
        c = a2 @ a1
        for b in range(_BT):
            r = b * _C
            W = W.at[br, r:r + _C, 2 * b:2 * b + 2].set(a1)
            W = W.at[br, _R + r:_R + r + _C, 2 * b:2 * b + 2].set(c)
            B2K = B2K.at[br, r:r + _C, r:r + _C].set(b2)

    xr = x.reshape(2, 2 * _B, _N)

    v = pl.pallas_call(
        _main_kernel,
        grid=(_B // _BT,),
        in_specs=[
            pl.BlockSpec((2, 2 * _BT, _N), lambda i: (0, i, 0)),
            pl.BlockSpec((2, 2 * _R, 2 * _BT), lambda i: (0, 0, 0)),
            pl.BlockSpec((2, _R, _R), lambda i: (0, 0, 0)),
        ],
        out_specs=pl.BlockSpec((2, 1, _R, _K), lambda i: (0, i, 0, 0)),
        out_shape=jax.ShapeDtypeStruct((2, _B // _BT, _R, _K), jnp.float32),
    )(xr, W, B2K)

    reps = (v.reshape(2, _B // _BT, _BT, _C, _K)
             .transpose(1, 2, 0, 3, 4)
             .reshape(_B, 2 * _C * _K))

    logits = pl.pallas_call(
        _head_kernel,
        in_specs=[
            pl.BlockSpec((_B, 2 * _C * _K), lambda: (0, 0)),
            pl.BlockSpec((2 * _C * _K, 5), lambda: (0, 0)),
            pl.BlockSpec((1, 5), lambda: (0, 0)),
        ],
        out_specs=pl.BlockSpec((_B, 5), lambda: (0, 0)),
        out_shape=jax.ShapeDtypeStruct((_B, 5), jnp.float32),
    )(reps, Wm.T, bm.reshape(1, 5))
    return logits


# speedup vs baseline: 153.9240x; 1.0515x over previous
import jax
import jax.numpy as jnp
from jax.experimental import pallas as pl

_B = 128
_N = 8192
_C = 25
_K = 5
_BT = 8
_R = _BT * _C
_LANES = 128


def _merge(A, B, cap=_K):
    la, lb = len(A), len(B)
    out = []
    for j in range(min(la + lb, cap)):
        terms = []
        if j < lb:
            terms.append(B[j])
        if j < la:
            terms.append(A[j])
        for i in range(1, j + 1):
            if i - 1 < la and j - i < lb:
                terms.append(jnp.minimum(A[i - 1], B[j - i]))
        acc = terms[0]
        for t in terms[1:]:
            acc = jnp.maximum(acc, t)
        out.append(acc)
    return out


def _pairtree(xs, op):
    while len(xs) > 1:
        nxt = [op(xs[2 * i], xs[2 * i + 1]) for i in range(len(xs) // 2)]
        if len(xs) % 2:
            nxt.append(xs[-1])
        xs = nxt
    return xs[0]


def _ce(M, i, j):
    hi = jnp.maximum(M[i], M[j])
    M[j] = jnp.minimum(M[i], M[j])
    M[i] = hi


def _merge44_5(A, B):
    M = [jnp.maximum(A[i], B[3 - i]) for i in range(4)]
    H = [jnp.minimum(A[i], B[3 - i]) for i in range(4)]
    _ce(M, 0, 2)
    _ce(M, 1, 3)
    _ce(M, 0, 1)
    _ce(M, 2, 3)
    fifth = jnp.maximum(jnp.maximum(H[0], H[1]), jnp.maximum(H[2], H[3]))
    return M + [fifth]


def _merge55(A, B):
    M = [jnp.maximum(A[i], B[4 - i]) for i in range(5)]
    _ce(M, 0, 4)
    _ce(M, 1, 3)
    _ce(M, 2, 4)
    _ce(M, 1, 2)
    _ce(M, 3, 4)
    return M


def _top5_rows(q):
    level = [[q[:, c * _LANES:(c + 1) * _LANES]] for c in range(_N // _LANES)]
    while len(level[0]) < 4:
        level = [_merge(level[2 * i], level[2 * i + 1])
                 for i in range(len(level) // 2)]
    level = [_merge44_5(level[2 * i], level[2 * i + 1])
             for i in range(len(level) // 2)]
    while len(level) > 1:
        level = [_merge55(level[2 * i], level[2 * i + 1])
                 for i in range(len(level) // 2)]
    T = level[0]
    s = _LANES // 2
    while s >= 1:
        T = _merge55(T, [jnp.roll(t, s, axis=1) for t in T])
        s //= 2
    return [t[:, 0:1] for t in T]


def _main_kernel(x_ref, w_ref, b2k_ref, v_ref):
    oh = [(jax.lax.broadcasted_iota(jnp.int32, (1, _K), 1) == c
           ).astype(jnp.float32) for c in range(_K)]
    for br in range(2):
        x16 = x_ref[br]
        g = jnp.dot(w_ref[br], x16,
                    preferred_element_type=jnp.float32)
        h = g[0:_R, :]
        q = g[_R:2 * _R, :]
        hm = _pairtree([h[:, c * _LANES:(c + 1) * _LANES]
                        for c in range(_N // _LANES)], jnp.maximum)
        s = _LANES // 2
        while s >= 1:
            hm = jnp.maximum(hm, jnp.roll(hm, s, axis=1))
            s //= 2
        m = hm[:, 0:1]
        bias = jnp.dot(b2k_ref[br], m,
                       preferred_element_type=jnp.float32)
        tops = _top5_rows(q)
        v = bias + tops[0] * oh[0]
        for t in range(1, _K):
            v = v + tops[t] * oh[t]
        v_ref[br, 0] = v


def _head_kernel(v_ref, w_ref, b_ref, o_ref):
    o_ref[...] = (jnp.dot(v_ref[...], w_ref[...],
                          preferred_element_type=jnp.float32)
                  + b_ref[...])


def kernel(x, h0_a1, h0_a2, h0_b2, h1_a1, h1_a2, h1_b2, Wm, bm):
    W = jnp.zeros((2, 2 * _R, 2 * _BT), jnp.float32)
    B2K = jnp.zeros((2, _R, _R), jnp.float32)
    for br, (a1, a2, b2) in enumerate(((h0_a1, h0_a2, h0_b2),
                                       (h1_a1, h1_a2, h1_b2))):
        c = a2 @ a1
        for b in range(_BT):
            r = b * _C
            W = W.at[br, r:r + _C, 2 * b:2 * b + 2].set(a1)
            W = W.at[br, _R + r:_R + r + _C, 2 * b:2 * b + 2].set(c)
            B2K = B2K.at[br, r:r + _C, r:r + _C].set(b2)

    xr = x.reshape(2, 2 * _B, _N)

    v = pl.pallas_call(
        _main_kernel,
        grid=(_B // _BT,),
        in_specs=[
            pl.BlockSpec((2, 2 * _BT, _N), lambda i: (0, i, 0)),
            pl.BlockSpec((2, 2 * _R, 2 * _BT), lambda i: (0, 0, 0)),
            pl.BlockSpec((2, _R, _R), lambda i: (0, 0, 0)),
        ],
        out_specs=pl.BlockSpec((2, 1, _R, _K), lambda i: (0, i, 0, 0)),
        out_shape=jax.ShapeDtypeStruct((2, _B // _BT, _R, _K), jnp.float32),
    )(xr, W, B2K)

    reps = (v.reshape(2, _B // _BT, _BT, _C, _K)
             .transpose(1, 2, 0, 3, 4)
             .reshape(_B, 2 * _C * _K))

    logits = pl.pallas_call(
        _head_kernel,
        in_specs=[
            pl.BlockSpec((_B, 2 * _C * _K), lambda: (0, 0)),
            pl.BlockSpec((2 * _C * _K, 5), lambda: (0, 0)),
            pl.BlockSpec((1, 5), lambda: (0, 0)),
        ],
        out_specs=pl.BlockSpec((_B, 5), lambda: (0, 0)),
        out_shape=jax.ShapeDtypeStruct((_B, 5), jnp.float32),
    )(reps, Wm.T, bm.reshape(1, 5))
    return logits
